# baseline (speedup 1.0000x reference)
import functools

import jax
import jax.numpy as jnp
import numpy as np
from jax import lax
from jax.experimental import pallas as pl
from jax.experimental.pallas import tpu as pltpu
from jax.experimental.pallas import tpu_sc as plsc

N = 10000
D = 128
N_CLS = 64
E = 320000

NC = 2
NS = 16
NW = NC * NS
C = 128
CH0 = 160
CH1 = 0
NCH = NS * (CH0 + CH1)
E_PAD = NCH * C
CHUNKS = CH0 + CH1
N_PAD = 10240
RPT = N_PAD // NS

RB = 1024
GRID = -(-N // RB)


def _sc_agg_body(with_deg, *refs):
    if with_deg:
        (h_hbm, sd_hbm, z_hbm, z1_hbm, acc_out, deg_out,
         sd0, sd1, sd2, sd3, rows0, rows1, deg_v, acc_sh,
         isem0, isem1, isem2, isem3, gsem0, gsem1, ssem0, ssem1) = refs
    else:
        (h_hbm, sd_hbm, z_hbm, acc_out,
         sd0, sd1, sd2, sd3, rows0, rows1, acc_sh,
         isem0, isem1, isem2, isem3, gsem0, gsem1, ssem0, ssem1) = refs
    sd = (sd0, sd1, sd2, sd3)
    isem = (isem0, isem1, isem2, isem3)
    rows = (rows0, rows1)
    gsem = (gsem0, gsem1)
    ssem = (ssem0, ssem1)
    SRC = np.int32(0)
    DST = np.int32(1)

    c = lax.axis_index("c")
    s = lax.axis_index("s")
    wid = s * jnp.int32(NC) + c
    r0 = s * jnp.int32(RPT)
    nch = jnp.where(c == jnp.int32(0), jnp.int32(CH0), jnp.int32(CH1))
    w0 = jnp.where(c == jnp.int32(0), s * jnp.int32(CH0),
                   jnp.int32(NS * CH0) + s * jnp.int32(CH1))

    pltpu.sync_copy(z_hbm, acc_sh.at[pl.ds(r0, RPT)])
    if with_deg:
        pltpu.sync_copy(z1_hbm, deg_v)
        ones16 = jnp.full((16,), 1.0, jnp.float32)

        def do_deg(q):
            for j in range(C // 16):
                idx16 = sd[q][1, pl.ds(j * 16, 16)]
                plsc.addupdate_scatter(deg_v, [idx16], ones16)
    else:
        def do_deg(q):
            pass

    def load_sd(g, q, sem):
        pltpu.async_copy(sd_hbm.at[pl.ds((w0 + g) * 2, 2)], sd[q], sem)

    plsc.subcore_barrier()

    @pl.when(nch > jnp.int32(0))
    def _():
        load_sd(jnp.int32(0), 0, isem0)
        load_sd(jnp.int32(1), 1, isem1)

    @pl.loop(jnp.int32(0), nch, step=4)
    def _(t):
        for b in range(4):
            p, q = b % 2, b
            g = t + jnp.int32(b)
            qn = (b + 2) % 4

            if b < 2:
                @pl.when(t > jnp.int32(0))
                def _():
                    pltpu.make_async_copy(rows[p], acc_sh.at[sd[qn].at[DST]],
                                          ssem[p]).wait()
            else:
                pltpu.make_async_copy(rows[p], acc_sh.at[sd[qn].at[DST]],
                                      ssem[p]).wait()

            if b < 2:
                load_sd(g + 2, qn, isem[qn])
            else:
                @pl.when(g + 2 < nch)
                def _():
                    load_sd(g + 2, qn, isem[qn])

            pltpu.make_async_copy(sd_hbm.at[pl.ds((w0 + g) * 2, 2)], sd[q],
                                  isem[q]).wait()
            pltpu.async_copy(h_hbm.at[sd[q].at[SRC]], rows[p], gsem[p])
            pltpu.make_async_copy(h_hbm.at[sd[q].at[SRC]], rows[p],
                                  gsem[p]).wait()
            pltpu.async_copy(rows[p], acc_sh.at[sd[q].at[DST]], ssem[p],
                             add=True)
            do_deg(q)

    @pl.when(nch > jnp.int32(0))
    def _():
        pltpu.make_async_copy(rows0, acc_sh.at[sd2.at[DST]], ssem0).wait()
        pltpu.make_async_copy(rows1, acc_sh.at[sd3.at[DST]], ssem1).wait()

    plsc.subcore_barrier()
    o0 = c * jnp.int32(N_PAD) + r0
    pltpu.sync_copy(acc_sh.at[pl.ds(r0, RPT)], acc_out.at[pl.ds(o0, RPT)])
    if with_deg:
        pltpu.sync_copy(deg_v, deg_out.at[wid])


def _make_sc_agg(with_deg):
    mesh = plsc.VectorSubcoreMesh(core_axis_name="c", subcore_axis_name="s")
    out_type = [jax.ShapeDtypeStruct((NC * N_PAD, D), jnp.float32)]
    scratch = [pltpu.VMEM((2, C), jnp.int32) for _ in range(4)] + [
        pltpu.VMEM((C, D), jnp.float32),
        pltpu.VMEM((C, D), jnp.float32),
    ]
    if with_deg:
        out_type.append(jax.ShapeDtypeStruct((NW, N_PAD), jnp.float32))
        scratch.append(pltpu.VMEM((N_PAD,), jnp.float32))
    return pl.kernel(
        functools.partial(_sc_agg_body, with_deg),
        out_type=tuple(out_type) if with_deg else out_type[0],
        mesh=mesh,
        scratch_types=scratch + [pltpu.VMEM_SHARED((N_PAD, D), jnp.float32)]
        + [pltpu.SemaphoreType.DMA] * 8,
        compiler_params=pltpu.CompilerParams(needs_layout_passes=False)
        if with_deg else None,
    )


_sc_agg_deg = _make_sc_agg(True)
_sc_agg = _make_sc_agg(False)


def _mm_t(a, b):
    return lax.dot_general(a, b, (((1,), (1,)), ((), ())),
                           preferred_element_type=jnp.float32)


def _tc_layer1_body(p_ref, dp_ref, h_ref, wl_ref, bl_ref, wr_ref,
                    o_ref, inv_ref):
    agg = p_ref[0] + p_ref[1]
    deg = lax.dot_general(dp_ref[...], jnp.ones((NW, D), jnp.float32),
                          (((0,), (0,)), ((), ())),
                          precision=lax.Precision.HIGHEST,
                          preferred_element_type=jnp.float32)
    inv = 1.0 / jnp.maximum(deg, 1.0)
    mean = agg * inv
    z = _mm_t(mean, wl_ref[...]) + bl_ref[...] + _mm_t(h_ref[...], wr_ref[...])
    o_ref[...] = jnp.maximum(z, 0.0)
    inv_ref[...] = inv


def _tc_layer_body(p_ref, inv_ref, h_ref, wl_ref, bl_ref, wr_ref, o_ref):
    mean = (p_ref[0] + p_ref[1]) * inv_ref[...]
    z = _mm_t(mean, wl_ref[...]) + bl_ref[...] + _mm_t(h_ref[...], wr_ref[...])
    o_ref[...] = jnp.maximum(z, 0.0)


def _tc_final_body(p_ref, inv_ref, h_ref, wl_ref, bl_ref, wr_ref,
                   wfc_ref, bfc_ref, emb_ref, out_ref):
    mean = (p_ref[0] + p_ref[1]) * inv_ref[...]
    z = _mm_t(mean, wl_ref[...]) + bl_ref[...] + _mm_t(h_ref[...], wr_ref[...])
    h = jnp.maximum(z, 0.0)
    nrm = jnp.sqrt(jnp.sum(h * h, axis=1, keepdims=True))
    emb = h / jnp.maximum(nrm, 1e-12)
    emb_ref[...] = emb
    out_ref[...] = _mm_t(emb, wfc_ref[...]) + bfc_ref[...]


_I0 = np.int32(0)
_SPEC_P = pl.BlockSpec((NC, RB, D), lambda i: (_I0, i, _I0))
_SPEC_DP = pl.BlockSpec((NW, RB), lambda i: (_I0, i))
_SPEC_H = pl.BlockSpec((RB, D), lambda i: (i, _I0))
_SPEC_W = pl.BlockSpec((D, D), lambda i: (_I0, _I0))
_SPEC_B = pl.BlockSpec((1, D), lambda i: (_I0, _I0))

_tc_layer1 = pl.pallas_call(
    _tc_layer1_body,
    grid=(GRID,),
    in_specs=[_SPEC_P, _SPEC_DP, _SPEC_H, _SPEC_W, _SPEC_B, _SPEC_W],
    out_specs=[_SPEC_H, _SPEC_H],
    out_shape=[jax.ShapeDtypeStruct((N, D), jnp.float32),
               jax.ShapeDtypeStruct((N, D), jnp.float32)],
)

_tc_layer = pl.pallas_call(
    _tc_layer_body,
    grid=(GRID,),
    in_specs=[_SPEC_P, _SPEC_H, _SPEC_H, _SPEC_W, _SPEC_B, _SPEC_W],
    out_specs=_SPEC_H,
    out_shape=jax.ShapeDtypeStruct((N, D), jnp.float32),
)

_tc_final = pl.pallas_call(
    _tc_final_body,
    grid=(GRID,),
    in_specs=[_SPEC_P, _SPEC_H, _SPEC_H, _SPEC_W, _SPEC_B, _SPEC_W,
              pl.BlockSpec((N_CLS, D), lambda i: (_I0, _I0)),
              pl.BlockSpec((1, N_CLS), lambda i: (_I0, _I0))],
    out_specs=[_SPEC_H, pl.BlockSpec((RB, N_CLS), lambda i: (i, _I0))],
    out_shape=[jax.ShapeDtypeStruct((N, D), jnp.float32),
               jax.ShapeDtypeStruct((N, N_CLS), jnp.float32)],
)


def kernel(x, edge_index, W1l, b1l, W1r, W2l, b2l, W2r, W3l, b3l, W3r,
           Wfc, bfc):
    src = edge_index[0].astype(jnp.int32)
    dst = edge_index[1].astype(jnp.int32)
    pad = E_PAD - E
    src_p = jnp.concatenate([src, jnp.zeros((pad,), jnp.int32)])
    dst_p = jnp.concatenate([dst, jnp.full((pad,), N, jnp.int32)])
    sd = jnp.stack([src_p.reshape(NCH, C), dst_p.reshape(NCH, C)], axis=1)
    sd = sd.reshape(NCH * 2, C)

    zeros = jnp.zeros((RPT, D), jnp.float32)
    zeros1 = jnp.zeros((N_PAD,), jnp.float32)

    b1 = b1l.reshape(1, D)
    b2 = b2l.reshape(1, D)
    b3 = b3l.reshape(1, D)
    bf = bfc.reshape(1, N_CLS)

    p1, dp = _sc_agg_deg(x, sd, zeros, zeros1)
    p1 = p1.reshape(NC, N_PAD, D)
    h1, invdeg = _tc_layer1(p1, dp, x, W1l, b1, W1r)
    p2 = _sc_agg(h1, sd, zeros).reshape(NC, N_PAD, D)
    h2 = _tc_layer(p2, invdeg, h1, W2l, b2, W2r)
    p3 = _sc_agg(h2, sd, zeros).reshape(NC, N_PAD, D)
    emb, out = _tc_final(p3, invdeg, h2, W3l, b3, W3r, Wfc, bf)
    return (emb, out)


# speedup vs baseline: 1.2409x; 1.2409x over previous
import functools

import jax
import jax.numpy as jnp
import numpy as np
from jax import lax
from jax.experimental import pallas as pl
from jax.experimental.pallas import tpu as pltpu
from jax.experimental.pallas import tpu_sc as plsc

N = 10000
D = 128
N_CLS = 64
E = 320000

NC = 2
NS = 16
NW = NC * NS
C = 128
CH0 = 112
CH1 = 48
NCH = NS * (CH0 + CH1)
E_PAD = NCH * C
CHUNKS = CH0 + CH1
N_PAD = 10240
RPT = N_PAD // NS

RB = 1024
GRID = -(-N // RB)


def _sc_agg_body(with_deg, *refs):
    if with_deg:
        (h_hbm, sd_hbm, z_hbm, z1_hbm, acc_out, deg_out,
         sd0, sd1, sd2, sd3, rows0, rows1, deg_v, acc_sh,
         isem0, isem1, isem2, isem3, gsem0, gsem1, ssem0, ssem1) = refs
    else:
        (h_hbm, sd_hbm, z_hbm, acc_out,
         sd0, sd1, sd2, sd3, rows0, rows1, acc_sh,
         isem0, isem1, isem2, isem3, gsem0, gsem1, ssem0, ssem1) = refs
    sd = (sd0, sd1, sd2, sd3)
    isem = (isem0, isem1, isem2, isem3)
    rows = (rows0, rows1)
    gsem = (gsem0, gsem1)
    ssem = (ssem0, ssem1)
    SRC = np.int32(0)
    DST = np.int32(1)

    c = lax.axis_index("c")
    s = lax.axis_index("s")
    wid = s * jnp.int32(NC) + c
    r0 = s * jnp.int32(RPT)
    nch = jnp.where(c == jnp.int32(0), jnp.int32(CH0), jnp.int32(CH1))
    w0 = jnp.where(c == jnp.int32(0), s * jnp.int32(CH0),
                   jnp.int32(NS * CH0) + s * jnp.int32(CH1))

    pltpu.sync_copy(z_hbm, acc_sh.at[pl.ds(r0, RPT)])
    if with_deg:
        pltpu.sync_copy(z1_hbm, deg_v)
        ones16 = jnp.full((16,), 1.0, jnp.float32)

        def do_deg(q):
            for j in range(C // 16):
                idx16 = sd[q][1, pl.ds(j * 16, 16)]
                plsc.addupdate_scatter(deg_v, [idx16], ones16)
    else:
        def do_deg(q):
            pass

    def load_sd(g, q, sem):
        pltpu.async_copy(sd_hbm.at[pl.ds((w0 + g) * 2, 2)], sd[q], sem)

    plsc.subcore_barrier()

    @pl.when(nch > jnp.int32(0))
    def _():
        load_sd(jnp.int32(0), 0, isem0)
        load_sd(jnp.int32(1), 1, isem1)

    @pl.loop(jnp.int32(0), nch, step=4)
    def _(t):
        for b in range(4):
            p, q = b % 2, b
            g = t + jnp.int32(b)
            qn = (b + 2) % 4

            if b < 2:
                @pl.when(t > jnp.int32(0))
                def _():
                    pltpu.make_async_copy(rows[p], acc_sh.at[sd[qn].at[DST]],
                                          ssem[p]).wait()
            else:
                pltpu.make_async_copy(rows[p], acc_sh.at[sd[qn].at[DST]],
                                      ssem[p]).wait()

            if b < 2:
                load_sd(g + 2, qn, isem[qn])
            else:
                @pl.when(g + 2 < nch)
                def _():
                    load_sd(g + 2, qn, isem[qn])

            pltpu.make_async_copy(sd_hbm.at[pl.ds((w0 + g) * 2, 2)], sd[q],
                                  isem[q]).wait()
            pltpu.async_copy(h_hbm.at[sd[q].at[SRC]], rows[p], gsem[p])
            pltpu.make_async_copy(h_hbm.at[sd[q].at[SRC]], rows[p],
                                  gsem[p]).wait()
            pltpu.async_copy(rows[p], acc_sh.at[sd[q].at[DST]], ssem[p],
                             add=True)
            do_deg(q)

    @pl.when(nch > jnp.int32(0))
    def _():
        pltpu.make_async_copy(rows0, acc_sh.at[sd2.at[DST]], ssem0).wait()
        pltpu.make_async_copy(rows1, acc_sh.at[sd3.at[DST]], ssem1).wait()

    plsc.subcore_barrier()
    o0 = c * jnp.int32(N_PAD) + r0
    pltpu.sync_copy(acc_sh.at[pl.ds(r0, RPT)], acc_out.at[pl.ds(o0, RPT)])
    if with_deg:
        pltpu.sync_copy(deg_v, deg_out.at[wid])


def _make_sc_agg(with_deg):
    mesh = plsc.VectorSubcoreMesh(core_axis_name="c", subcore_axis_name="s")
    out_type = [jax.ShapeDtypeStruct((NC * N_PAD, D), jnp.float32)]
    scratch = [pltpu.VMEM((2, C), jnp.int32) for _ in range(4)] + [
        pltpu.VMEM((C, D), jnp.float32),
        pltpu.VMEM((C, D), jnp.float32),
    ]
    if with_deg:
        out_type.append(jax.ShapeDtypeStruct((NW, N_PAD), jnp.float32))
        scratch.append(pltpu.VMEM((N_PAD,), jnp.float32))
    return pl.kernel(
        functools.partial(_sc_agg_body, with_deg),
        out_type=tuple(out_type) if with_deg else out_type[0],
        mesh=mesh,
        scratch_types=scratch + [pltpu.VMEM_SHARED((N_PAD, D), jnp.float32)]
        + [pltpu.SemaphoreType.DMA] * 8,
        compiler_params=pltpu.CompilerParams(needs_layout_passes=False)
        if with_deg else None,
    )


_sc_agg_deg = _make_sc_agg(True)
_sc_agg = _make_sc_agg(False)


def _mm_t(a, b):
    return lax.dot_general(a, b, (((1,), (1,)), ((), ())),
                           preferred_element_type=jnp.float32)


def _tc_layer1_body(p_ref, dp_ref, h_ref, wl_ref, bl_ref, wr_ref,
                    o_ref, inv_ref):
    agg = p_ref[0] + p_ref[1]
    deg = lax.dot_general(dp_ref[...], jnp.ones((NW, D), jnp.float32),
                          (((0,), (0,)), ((), ())),
                          precision=lax.Precision.HIGHEST,
                          preferred_element_type=jnp.float32)
    inv = 1.0 / jnp.maximum(deg, 1.0)
    mean = agg * inv
    z = _mm_t(mean, wl_ref[...]) + bl_ref[...] + _mm_t(h_ref[...], wr_ref[...])
    o_ref[...] = jnp.maximum(z, 0.0)
    inv_ref[...] = inv


def _tc_layer_body(p_ref, inv_ref, h_ref, wl_ref, bl_ref, wr_ref, o_ref):
    mean = (p_ref[0] + p_ref[1]) * inv_ref[...]
    z = _mm_t(mean, wl_ref[...]) + bl_ref[...] + _mm_t(h_ref[...], wr_ref[...])
    o_ref[...] = jnp.maximum(z, 0.0)


def _tc_final_body(p_ref, inv_ref, h_ref, wl_ref, bl_ref, wr_ref,
                   wfc_ref, bfc_ref, emb_ref, out_ref):
    mean = (p_ref[0] + p_ref[1]) * inv_ref[...]
    z = _mm_t(mean, wl_ref[...]) + bl_ref[...] + _mm_t(h_ref[...], wr_ref[...])
    h = jnp.maximum(z, 0.0)
    nrm = jnp.sqrt(jnp.sum(h * h, axis=1, keepdims=True))
    emb = h / jnp.maximum(nrm, 1e-12)
    emb_ref[...] = emb
    out_ref[...] = _mm_t(emb, wfc_ref[...]) + bfc_ref[...]


_I0 = np.int32(0)
_SPEC_P = pl.BlockSpec((NC, RB, D), lambda i: (_I0, i, _I0))
_SPEC_DP = pl.BlockSpec((NW, RB), lambda i: (_I0, i))
_SPEC_H = pl.BlockSpec((RB, D), lambda i: (i, _I0))
_SPEC_W = pl.BlockSpec((D, D), lambda i: (_I0, _I0))
_SPEC_B = pl.BlockSpec((1, D), lambda i: (_I0, _I0))

_tc_layer1 = pl.pallas_call(
    _tc_layer1_body,
    grid=(GRID,),
    in_specs=[_SPEC_P, _SPEC_DP, _SPEC_H, _SPEC_W, _SPEC_B, _SPEC_W],
    out_specs=[_SPEC_H, _SPEC_H],
    out_shape=[jax.ShapeDtypeStruct((N, D), jnp.float32),
               jax.ShapeDtypeStruct((N, D), jnp.float32)],
)

_tc_layer = pl.pallas_call(
    _tc_layer_body,
    grid=(GRID,),
    in_specs=[_SPEC_P, _SPEC_H, _SPEC_H, _SPEC_W, _SPEC_B, _SPEC_W],
    out_specs=_SPEC_H,
    out_shape=jax.ShapeDtypeStruct((N, D), jnp.float32),
)

_tc_final = pl.pallas_call(
    _tc_final_body,
    grid=(GRID,),
    in_specs=[_SPEC_P, _SPEC_H, _SPEC_H, _SPEC_W, _SPEC_B, _SPEC_W,
              pl.BlockSpec((N_CLS, D), lambda i: (_I0, _I0)),
              pl.BlockSpec((1, N_CLS), lambda i: (_I0, _I0))],
    out_specs=[_SPEC_H, pl.BlockSpec((RB, N_CLS), lambda i: (i, _I0))],
    out_shape=[jax.ShapeDtypeStruct((N, D), jnp.float32),
               jax.ShapeDtypeStruct((N, N_CLS), jnp.float32)],
)


def kernel(x, edge_index, W1l, b1l, W1r, W2l, b2l, W2r, W3l, b3l, W3r,
           Wfc, bfc):
    src = edge_index[0].astype(jnp.int32)
    dst = edge_index[1].astype(jnp.int32)
    pad = E_PAD - E
    src_p = jnp.concatenate([src, jnp.zeros((pad,), jnp.int32)])
    dst_p = jnp.concatenate([dst, jnp.full((pad,), N, jnp.int32)])
    sd = jnp.stack([src_p.reshape(NCH, C), dst_p.reshape(NCH, C)], axis=1)
    sd = sd.reshape(NCH * 2, C)

    zeros = jnp.zeros((RPT, D), jnp.float32)
    zeros1 = jnp.zeros((N_PAD,), jnp.float32)

    b1 = b1l.reshape(1, D)
    b2 = b2l.reshape(1, D)
    b3 = b3l.reshape(1, D)
    bf = bfc.reshape(1, N_CLS)

    p1, dp = _sc_agg_deg(x, sd, zeros, zeros1)
    p1 = p1.reshape(NC, N_PAD, D)
    h1, invdeg = _tc_layer1(p1, dp, x, W1l, b1, W1r)
    p2 = _sc_agg(h1, sd, zeros).reshape(NC, N_PAD, D)
    h2 = _tc_layer(p2, invdeg, h1, W2l, b2, W2r)
    p3 = _sc_agg(h2, sd, zeros).reshape(NC, N_PAD, D)
    emb, out = _tc_final(p3, invdeg, h2, W3l, b3, W3r, Wfc, bf)
    return (emb, out)


# speedup vs baseline: 1.5422x; 1.2428x over previous
import functools

import jax
import jax.numpy as jnp
import numpy as np
from jax import lax
from jax.experimental import pallas as pl
from jax.experimental.pallas import tpu as pltpu
from jax.experimental.pallas import tpu_sc as plsc

N = 10000
D = 128
N_CLS = 64
E = 320000

NC = 2
NS = 16
NW = NC * NS
C = 128
CH0 = 144
CH1 = 16
NCH = NS * (CH0 + CH1)
E_PAD = NCH * C
CHUNKS = CH0 + CH1
N_PAD = 10240
RPT = N_PAD // NS

RB = 1024
GRID = -(-N // RB)


def _sc_agg_body(with_deg, *refs):
    if with_deg:
        (h_hbm, sd_hbm, z_hbm, z1_hbm, acc_out, deg_out,
         sd0, sd1, sd2, sd3, rows0, rows1, deg_v, acc_sh,
         isem0, isem1, isem2, isem3, gsem0, gsem1, ssem0, ssem1) = refs
    else:
        (h_hbm, sd_hbm, z_hbm, acc_out,
         sd0, sd1, sd2, sd3, rows0, rows1, acc_sh,
         isem0, isem1, isem2, isem3, gsem0, gsem1, ssem0, ssem1) = refs
    sd = (sd0, sd1, sd2, sd3)
    isem = (isem0, isem1, isem2, isem3)
    rows = (rows0, rows1)
    gsem = (gsem0, gsem1)
    ssem = (ssem0, ssem1)
    SRC = np.int32(0)
    DST = np.int32(1)

    c = lax.axis_index("c")
    s = lax.axis_index("s")
    wid = s * jnp.int32(NC) + c
    r0 = s * jnp.int32(RPT)
    nch = jnp.where(c == jnp.int32(0), jnp.int32(CH0), jnp.int32(CH1))
    w0 = jnp.where(c == jnp.int32(0), s * jnp.int32(CH0),
                   jnp.int32(NS * CH0) + s * jnp.int32(CH1))

    pltpu.sync_copy(z_hbm, acc_sh.at[pl.ds(r0, RPT)])
    if with_deg:
        pltpu.sync_copy(z1_hbm, deg_v)
        ones16 = jnp.full((16,), 1.0, jnp.float32)

        def do_deg(q):
            for j in range(C // 16):
                idx16 = sd[q][1, pl.ds(j * 16, 16)]
                plsc.addupdate_scatter(deg_v, [idx16], ones16)
    else:
        def do_deg(q):
            pass

    def load_sd(g, q, sem):
        pltpu.async_copy(sd_hbm.at[pl.ds((w0 + g) * 2, 2)], sd[q], sem)

    plsc.subcore_barrier()

    @pl.when(nch > jnp.int32(0))
    def _():
        load_sd(jnp.int32(0), 0, isem0)
        load_sd(jnp.int32(1), 1, isem1)

    @pl.loop(jnp.int32(0), nch, step=4)
    def _(t):
        for b in range(4):
            p, q = b % 2, b
            g = t + jnp.int32(b)
            qn = (b + 2) % 4

            if b < 2:
                @pl.when(t > jnp.int32(0))
                def _():
                    pltpu.make_async_copy(rows[p], acc_sh.at[sd[qn].at[DST]],
                                          ssem[p]).wait()
            else:
                pltpu.make_async_copy(rows[p], acc_sh.at[sd[qn].at[DST]],
                                      ssem[p]).wait()

            if b < 2:
                load_sd(g + 2, qn, isem[qn])
            else:
                @pl.when(g + 2 < nch)
                def _():
                    load_sd(g + 2, qn, isem[qn])

            pltpu.make_async_copy(sd_hbm.at[pl.ds((w0 + g) * 2, 2)], sd[q],
                                  isem[q]).wait()
            pltpu.async_copy(h_hbm.at[sd[q].at[SRC]], rows[p], gsem[p])
            pltpu.make_async_copy(h_hbm.at[sd[q].at[SRC]], rows[p],
                                  gsem[p]).wait()
            pltpu.async_copy(rows[p], acc_sh.at[sd[q].at[DST]], ssem[p],
                             add=True)
            do_deg(q)

    @pl.when(nch > jnp.int32(0))
    def _():
        pltpu.make_async_copy(rows0, acc_sh.at[sd2.at[DST]], ssem0).wait()
        pltpu.make_async_copy(rows1, acc_sh.at[sd3.at[DST]], ssem1).wait()

    plsc.subcore_barrier()
    o0 = c * jnp.int32(N_PAD) + r0
    pltpu.sync_copy(acc_sh.at[pl.ds(r0, RPT)], acc_out.at[pl.ds(o0, RPT)])
    if with_deg:
        pltpu.sync_copy(deg_v, deg_out.at[wid])


def _make_sc_agg(with_deg):
    mesh = plsc.VectorSubcoreMesh(core_axis_name="c", subcore_axis_name="s")
    out_type = [jax.ShapeDtypeStruct((NC * N_PAD, D), jnp.float32)]
    scratch = [pltpu.VMEM((2, C), jnp.int32) for _ in range(4)] + [
        pltpu.VMEM((C, D), jnp.float32),
        pltpu.VMEM((C, D), jnp.float32),
    ]
    if with_deg:
        out_type.append(jax.ShapeDtypeStruct((NW, N_PAD), jnp.float32))
        scratch.append(pltpu.VMEM((N_PAD,), jnp.float32))
    return pl.kernel(
        functools.partial(_sc_agg_body, with_deg),
        out_type=tuple(out_type) if with_deg else out_type[0],
        mesh=mesh,
        scratch_types=scratch + [pltpu.VMEM_SHARED((N_PAD, D), jnp.float32)]
        + [pltpu.SemaphoreType.DMA] * 8,
        compiler_params=pltpu.CompilerParams(needs_layout_passes=False)
        if with_deg else None,
    )


_sc_agg_deg = _make_sc_agg(True)
_sc_agg = _make_sc_agg(False)


def _mm_t(a, b):
    return lax.dot_general(a, b, (((1,), (1,)), ((), ())),
                           preferred_element_type=jnp.float32)


def _tc_layer1_body(p_ref, dp_ref, h_ref, wl_ref, bl_ref, wr_ref,
                    o_ref, inv_ref):
    agg = p_ref[0] + p_ref[1]
    deg = lax.dot_general(dp_ref[...], jnp.ones((NW, D), jnp.float32),
                          (((0,), (0,)), ((), ())),
                          precision=lax.Precision.HIGHEST,
                          preferred_element_type=jnp.float32)
    inv = 1.0 / jnp.maximum(deg, 1.0)
    mean = agg * inv
    z = _mm_t(mean, wl_ref[...]) + bl_ref[...] + _mm_t(h_ref[...], wr_ref[...])
    o_ref[...] = jnp.maximum(z, 0.0)
    inv_ref[...] = inv


def _tc_layer_body(p_ref, inv_ref, h_ref, wl_ref, bl_ref, wr_ref, o_ref):
    mean = (p_ref[0] + p_ref[1]) * inv_ref[...]
    z = _mm_t(mean, wl_ref[...]) + bl_ref[...] + _mm_t(h_ref[...], wr_ref[...])
    o_ref[...] = jnp.maximum(z, 0.0)


def _tc_final_body(p_ref, inv_ref, h_ref, wl_ref, bl_ref, wr_ref,
                   wfc_ref, bfc_ref, emb_ref, out_ref):
    mean = (p_ref[0] + p_ref[1]) * inv_ref[...]
    z = _mm_t(mean, wl_ref[...]) + bl_ref[...] + _mm_t(h_ref[...], wr_ref[...])
    h = jnp.maximum(z, 0.0)
    nrm = jnp.sqrt(jnp.sum(h * h, axis=1, keepdims=True))
    emb = h / jnp.maximum(nrm, 1e-12)
    emb_ref[...] = emb
    out_ref[...] = _mm_t(emb, wfc_ref[...]) + bfc_ref[...]


_I0 = np.int32(0)
_SPEC_P = pl.BlockSpec((NC, RB, D), lambda i: (_I0, i, _I0))
_SPEC_DP = pl.BlockSpec((NW, RB), lambda i: (_I0, i))
_SPEC_H = pl.BlockSpec((RB, D), lambda i: (i, _I0))
_SPEC_W = pl.BlockSpec((D, D), lambda i: (_I0, _I0))
_SPEC_B = pl.BlockSpec((1, D), lambda i: (_I0, _I0))

_tc_layer1 = pl.pallas_call(
    _tc_layer1_body,
    grid=(GRID,),
    in_specs=[_SPEC_P, _SPEC_DP, _SPEC_H, _SPEC_W, _SPEC_B, _SPEC_W],
    out_specs=[_SPEC_H, _SPEC_H],
    out_shape=[jax.ShapeDtypeStruct((N, D), jnp.float32),
               jax.ShapeDtypeStruct((N, D), jnp.float32)],
)

_tc_layer = pl.pallas_call(
    _tc_layer_body,
    grid=(GRID,),
    in_specs=[_SPEC_P, _SPEC_H, _SPEC_H, _SPEC_W, _SPEC_B, _SPEC_W],
    out_specs=_SPEC_H,
    out_shape=jax.ShapeDtypeStruct((N, D), jnp.float32),
)

_tc_final = pl.pallas_call(
    _tc_final_body,
    grid=(GRID,),
    in_specs=[_SPEC_P, _SPEC_H, _SPEC_H, _SPEC_W, _SPEC_B, _SPEC_W,
              pl.BlockSpec((N_CLS, D), lambda i: (_I0, _I0)),
              pl.BlockSpec((1, N_CLS), lambda i: (_I0, _I0))],
    out_specs=[_SPEC_H, pl.BlockSpec((RB, N_CLS), lambda i: (i, _I0))],
    out_shape=[jax.ShapeDtypeStruct((N, D), jnp.float32),
               jax.ShapeDtypeStruct((N, N_CLS), jnp.float32)],
)


def kernel(x, edge_index, W1l, b1l, W1r, W2l, b2l, W2r, W3l, b3l, W3r,
           Wfc, bfc):
    src = edge_index[0].astype(jnp.int32)
    dst = edge_index[1].astype(jnp.int32)
    pad = E_PAD - E
    src_p = jnp.concatenate([src, jnp.zeros((pad,), jnp.int32)])
    dst_p = jnp.concatenate([dst, jnp.full((pad,), N, jnp.int32)])
    sd = jnp.stack([src_p.reshape(NCH, C), dst_p.reshape(NCH, C)], axis=1)
    sd = sd.reshape(NCH * 2, C)

    zeros = jnp.zeros((RPT, D), jnp.float32)
    zeros1 = jnp.zeros((N_PAD,), jnp.float32)

    b1 = b1l.reshape(1, D)
    b2 = b2l.reshape(1, D)
    b3 = b3l.reshape(1, D)
    bf = bfc.reshape(1, N_CLS)

    p1, dp = _sc_agg_deg(x, sd, zeros, zeros1)
    p1 = p1.reshape(NC, N_PAD, D)
    h1, invdeg = _tc_layer1(p1, dp, x, W1l, b1, W1r)
    p2 = _sc_agg(h1, sd, zeros).reshape(NC, N_PAD, D)
    h2 = _tc_layer(p2, invdeg, h1, W2l, b2, W2r)
    p3 = _sc_agg(h2, sd, zeros).reshape(NC, N_PAD, D)
    emb, out = _tc_final(p3, invdeg, h2, W3l, b3, W3r, Wfc, bf)
    return (emb, out)


# speedup vs baseline: 1.5627x; 1.0133x over previous
import functools

import jax
import jax.numpy as jnp
import numpy as np
from jax import lax
from jax.experimental import pallas as pl
from jax.experimental.pallas import tpu as pltpu
from jax.experimental.pallas import tpu_sc as plsc

N = 10000
D = 128
N_CLS = 64
E = 320000

NC = 2
NS = 16
NW = NC * NS
C = 128
CH0 = 152
CH1 = 8
NCH = NS * (CH0 + CH1)
E_PAD = NCH * C
CHUNKS = CH0 + CH1
N_PAD = 10240
RPT = N_PAD // NS

RB = 1024
GRID = -(-N // RB)


def _sc_agg_body(with_deg, *refs):
    if with_deg:
        (h_hbm, sd_hbm, z_hbm, z1_hbm, acc_out, deg_out,
         sd0, sd1, sd2, sd3, rows0, rows1, deg_v, acc_sh,
         isem0, isem1, isem2, isem3, gsem0, gsem1, ssem0, ssem1) = refs
    else:
        (h_hbm, sd_hbm, z_hbm, acc_out,
         sd0, sd1, sd2, sd3, rows0, rows1, acc_sh,
         isem0, isem1, isem2, isem3, gsem0, gsem1, ssem0, ssem1) = refs
    sd = (sd0, sd1, sd2, sd3)
    isem = (isem0, isem1, isem2, isem3)
    rows = (rows0, rows1)
    gsem = (gsem0, gsem1)
    ssem = (ssem0, ssem1)
    SRC = np.int32(0)
    DST = np.int32(1)

    c = lax.axis_index("c")
    s = lax.axis_index("s")
    wid = s * jnp.int32(NC) + c
    r0 = s * jnp.int32(RPT)
    nch = jnp.where(c == jnp.int32(0), jnp.int32(CH0), jnp.int32(CH1))
    w0 = jnp.where(c == jnp.int32(0), s * jnp.int32(CH0),
                   jnp.int32(NS * CH0) + s * jnp.int32(CH1))

    pltpu.sync_copy(z_hbm, acc_sh.at[pl.ds(r0, RPT)])
    if with_deg:
        pltpu.sync_copy(z1_hbm, deg_v)
        ones16 = jnp.full((16,), 1.0, jnp.float32)

        def do_deg(q):
            for j in range(C // 16):
                idx16 = sd[q][1, pl.ds(j * 16, 16)]
                plsc.addupdate_scatter(deg_v, [idx16], ones16)
    else:
        def do_deg(q):
            pass

    def load_sd(g, q, sem):
        pltpu.async_copy(sd_hbm.at[pl.ds((w0 + g) * 2, 2)], sd[q], sem)

    plsc.subcore_barrier()

    @pl.when(nch > jnp.int32(0))
    def _():
        load_sd(jnp.int32(0), 0, isem0)
        load_sd(jnp.int32(1), 1, isem1)

    @pl.loop(jnp.int32(0), nch, step=4)
    def _(t):
        for b in range(4):
            p, q = b % 2, b
            g = t + jnp.int32(b)
            qn = (b + 2) % 4

            if b < 2:
                @pl.when(t > jnp.int32(0))
                def _():
                    pltpu.make_async_copy(rows[p], acc_sh.at[sd[qn].at[DST]],
                                          ssem[p]).wait()
            else:
                pltpu.make_async_copy(rows[p], acc_sh.at[sd[qn].at[DST]],
                                      ssem[p]).wait()

            if b < 2:
                load_sd(g + 2, qn, isem[qn])
            else:
                @pl.when(g + 2 < nch)
                def _():
                    load_sd(g + 2, qn, isem[qn])

            pltpu.make_async_copy(sd_hbm.at[pl.ds((w0 + g) * 2, 2)], sd[q],
                                  isem[q]).wait()
            pltpu.async_copy(h_hbm.at[sd[q].at[SRC]], rows[p], gsem[p])
            pltpu.make_async_copy(h_hbm.at[sd[q].at[SRC]], rows[p],
                                  gsem[p]).wait()
            pltpu.async_copy(rows[p], acc_sh.at[sd[q].at[DST]], ssem[p],
                             add=True)
            do_deg(q)

    @pl.when(nch > jnp.int32(0))
    def _():
        pltpu.make_async_copy(rows0, acc_sh.at[sd2.at[DST]], ssem0).wait()
        pltpu.make_async_copy(rows1, acc_sh.at[sd3.at[DST]], ssem1).wait()

    plsc.subcore_barrier()
    o0 = c * jnp.int32(N_PAD) + r0
    pltpu.sync_copy(acc_sh.at[pl.ds(r0, RPT)], acc_out.at[pl.ds(o0, RPT)])
    if with_deg:
        pltpu.sync_copy(deg_v, deg_out.at[wid])


def _make_sc_agg(with_deg):
    mesh = plsc.VectorSubcoreMesh(core_axis_name="c", subcore_axis_name="s")
    out_type = [jax.ShapeDtypeStruct((NC * N_PAD, D), jnp.float32)]
    scratch = [pltpu.VMEM((2, C), jnp.int32) for _ in range(4)] + [
        pltpu.VMEM((C, D), jnp.float32),
        pltpu.VMEM((C, D), jnp.float32),
    ]
    if with_deg:
        out_type.append(jax.ShapeDtypeStruct((NW, N_PAD), jnp.float32))
        scratch.append(pltpu.VMEM((N_PAD,), jnp.float32))
    return pl.kernel(
        functools.partial(_sc_agg_body, with_deg),
        out_type=tuple(out_type) if with_deg else out_type[0],
        mesh=mesh,
        scratch_types=scratch + [pltpu.VMEM_SHARED((N_PAD, D), jnp.float32)]
        + [pltpu.SemaphoreType.DMA] * 8,
        compiler_params=pltpu.CompilerParams(needs_layout_passes=False)
        if with_deg else None,
    )


_sc_agg_deg = _make_sc_agg(True)
_sc_agg = _make_sc_agg(False)


def _mm_t(a, b):
    return lax.dot_general(a, b, (((1,), (1,)), ((), ())),
                           preferred_element_type=jnp.float32)


def _tc_layer1_body(p_ref, dp_ref, h_ref, wl_ref, bl_ref, wr_ref,
                    o_ref, inv_ref):
    agg = p_ref[0] + p_ref[1]
    deg = lax.dot_general(dp_ref[...], jnp.ones((NW, D), jnp.float32),
                          (((0,), (0,)), ((), ())),
                          precision=lax.Precision.HIGHEST,
                          preferred_element_type=jnp.float32)
    inv = 1.0 / jnp.maximum(deg, 1.0)
    mean = agg * inv
    z = _mm_t(mean, wl_ref[...]) + bl_ref[...] + _mm_t(h_ref[...], wr_ref[...])
    o_ref[...] = jnp.maximum(z, 0.0)
    inv_ref[...] = inv


def _tc_layer_body(p_ref, inv_ref, h_ref, wl_ref, bl_ref, wr_ref, o_ref):
    mean = (p_ref[0] + p_ref[1]) * inv_ref[...]
    z = _mm_t(mean, wl_ref[...]) + bl_ref[...] + _mm_t(h_ref[...], wr_ref[...])
    o_ref[...] = jnp.maximum(z, 0.0)


def _tc_final_body(p_ref, inv_ref, h_ref, wl_ref, bl_ref, wr_ref,
                   wfc_ref, bfc_ref, emb_ref, out_ref):
    mean = (p_ref[0] + p_ref[1]) * inv_ref[...]
    z = _mm_t(mean, wl_ref[...]) + bl_ref[...] + _mm_t(h_ref[...], wr_ref[...])
    h = jnp.maximum(z, 0.0)
    nrm = jnp.sqrt(jnp.sum(h * h, axis=1, keepdims=True))
    emb = h / jnp.maximum(nrm, 1e-12)
    emb_ref[...] = emb
    out_ref[...] = _mm_t(emb, wfc_ref[...]) + bfc_ref[...]


_I0 = np.int32(0)
_SPEC_P = pl.BlockSpec((NC, RB, D), lambda i: (_I0, i, _I0))
_SPEC_DP = pl.BlockSpec((NW, RB), lambda i: (_I0, i))
_SPEC_H = pl.BlockSpec((RB, D), lambda i: (i, _I0))
_SPEC_W = pl.BlockSpec((D, D), lambda i: (_I0, _I0))
_SPEC_B = pl.BlockSpec((1, D), lambda i: (_I0, _I0))

_tc_layer1 = pl.pallas_call(
    _tc_layer1_body,
    grid=(GRID,),
    in_specs=[_SPEC_P, _SPEC_DP, _SPEC_H, _SPEC_W, _SPEC_B, _SPEC_W],
    out_specs=[_SPEC_H, _SPEC_H],
    out_shape=[jax.ShapeDtypeStruct((N, D), jnp.float32),
               jax.ShapeDtypeStruct((N, D), jnp.float32)],
)

_tc_layer = pl.pallas_call(
    _tc_layer_body,
    grid=(GRID,),
    in_specs=[_SPEC_P, _SPEC_H, _SPEC_H, _SPEC_W, _SPEC_B, _SPEC_W],
    out_specs=_SPEC_H,
    out_shape=jax.ShapeDtypeStruct((N, D), jnp.float32),
)

_tc_final = pl.pallas_call(
    _tc_final_body,
    grid=(GRID,),
    in_specs=[_SPEC_P, _SPEC_H, _SPEC_H, _SPEC_W, _SPEC_B, _SPEC_W,
              pl.BlockSpec((N_CLS, D), lambda i: (_I0, _I0)),
              pl.BlockSpec((1, N_CLS), lambda i: (_I0, _I0))],
    out_specs=[_SPEC_H, pl.BlockSpec((RB, N_CLS), lambda i: (i, _I0))],
    out_shape=[jax.ShapeDtypeStruct((N, D), jnp.float32),
               jax.ShapeDtypeStruct((N, N_CLS), jnp.float32)],
)


def kernel(x, edge_index, W1l, b1l, W1r, W2l, b2l, W2r, W3l, b3l, W3r,
           Wfc, bfc):
    src = edge_index[0].astype(jnp.int32)
    dst = edge_index[1].astype(jnp.int32)
    pad = E_PAD - E
    src_p = jnp.concatenate([src, jnp.zeros((pad,), jnp.int32)])
    dst_p = jnp.concatenate([dst, jnp.full((pad,), N, jnp.int32)])
    sd = jnp.stack([src_p.reshape(NCH, C), dst_p.reshape(NCH, C)], axis=1)
    sd = sd.reshape(NCH * 2, C)

    zeros = jnp.zeros((RPT, D), jnp.float32)
    zeros1 = jnp.zeros((N_PAD,), jnp.float32)

    b1 = b1l.reshape(1, D)
    b2 = b2l.reshape(1, D)
    b3 = b3l.reshape(1, D)
    bf = bfc.reshape(1, N_CLS)

    p1, dp = _sc_agg_deg(x, sd, zeros, zeros1)
    p1 = p1.reshape(NC, N_PAD, D)
    h1, invdeg = _tc_layer1(p1, dp, x, W1l, b1, W1r)
    p2 = _sc_agg(h1, sd, zeros).reshape(NC, N_PAD, D)
    h2 = _tc_layer(p2, invdeg, h1, W2l, b2, W2r)
    p3 = _sc_agg(h2, sd, zeros).reshape(NC, N_PAD, D)
    emb, out = _tc_final(p3, invdeg, h2, W3l, b3, W3r, Wfc, bf)
    return (emb, out)
